# baseline (speedup 1.0000x reference)
import jax
import jax.numpy as jnp
from jax.experimental import pallas as pl
from jax.experimental.pallas import tpu as pltpu

_N_CLASSES = 1000
_ROWS = 4096 * 26
_BLOCK_R = 1024


def _onehot_body(idx_ref, out_ref):
    idx = idx_ref[...]
    iota = jax.lax.broadcasted_iota(jnp.int32, (_BLOCK_R, _N_CLASSES), 1)
    out_ref[...] = (iota == idx).astype(jnp.float32)


def kernel(input):
    flat = input.reshape(_ROWS, 1).astype(jnp.int32)
    grid = _ROWS // _BLOCK_R
    out = pl.pallas_call(
        _onehot_body,
        grid=(grid,),
        in_specs=[pl.BlockSpec((_BLOCK_R, 1), lambda i: (i, 0))],
        out_specs=pl.BlockSpec((_BLOCK_R, _N_CLASSES), lambda i: (i, 0)),
        out_shape=jax.ShapeDtypeStruct((_ROWS, _N_CLASSES), jnp.float32),
        compiler_params=pltpu.CompilerParams(
            dimension_semantics=("arbitrary",),
        ),
    )(flat)
    return out.reshape(4096, 26, _N_CLASSES)


# speedup vs baseline: 1.4671x; 1.4671x over previous
import jax
import jax.numpy as jnp
from jax.experimental import pallas as pl
from jax.experimental.pallas import tpu as pltpu

_N_CLASSES = 1000
_B = 32


def _onehot_body(idx_ref, out_ref):
    idx = idx_ref[...]
    iota = jax.lax.broadcasted_iota(jnp.int32, (_B, 26, _N_CLASSES), 2)
    out_ref[...] = (iota == idx[:, :, None]).astype(jnp.float32)


def kernel(input):
    grid = 4096 // _B
    return pl.pallas_call(
        _onehot_body,
        grid=(grid,),
        in_specs=[pl.BlockSpec((_B, 26), lambda i: (i, 0))],
        out_specs=pl.BlockSpec((_B, 26, _N_CLASSES), lambda i: (i, 0, 0)),
        out_shape=jax.ShapeDtypeStruct((4096, 26, _N_CLASSES), jnp.float32),
        compiler_params=pltpu.CompilerParams(
            dimension_semantics=("arbitrary",),
        ),
    )(input)
